# Initial kernel scaffold; baseline (speedup 1.0000x reference)
import math

import jax
import jax.numpy as jnp
from jax import lax
from jax.experimental import pallas as pl
from jax.experimental.pallas import tpu as pltpu

N_BUCKETS = 32
MAX_DISTANCE = 128
N_HEADS = 16
S = 2048
M_PAD = 4352
EG_W = 4096


def _diag_table_kernel(biases_ref, out_ref):
    m = lax.broadcasted_iota(jnp.int32, (M_PAD, 1), 0)
    d = 2047 - m
    max_exact = N_BUCKETS // 2
    rp = jnp.maximum(d, 0)
    is_smol = rp < max_exact
    rp_f = jnp.maximum(rp, 1).astype(jnp.float32)
    val_if_large = max_exact + (
        jnp.log(rp_f / max_exact) / math.log(MAX_DISTANCE / max_exact)
        * (N_BUCKETS - max_exact)
    ).astype(jnp.int32)
    val_if_large = jnp.minimum(val_if_large, N_BUCKETS - 1)
    bucket = jnp.where(is_smol, rp, val_if_large)
    acc = jnp.zeros((M_PAD, N_HEADS), jnp.float32)
    for k in range(N_BUCKETS):
        acc = jnp.where(bucket == k, biases_ref[k : k + 1, :], acc)
    out_ref[:, :] = acc


def _make_diag_table(biases):
    return pl.pallas_call(
        _diag_table_kernel,
        out_shape=jax.ShapeDtypeStruct((M_PAD, N_HEADS), jnp.float32),
    )(biases)


def _add_bias_kernel(rdiag_ref, x_ref, out_ref, eg_ref):
    bi = pl.program_id(1)

    @pl.when(bi == 0)
    def _build_eg():
        for s in range(8):
            eg_ref[pl.ds(s, 1), :] = rdiag_ref[0, :, pl.ds(7 - s, EG_W)]

    shift = 2040 - 8 * bi
    bias = eg_ref[:, pl.ds(shift, S)]
    out_ref[0, 0] = x_ref[0, 0] + bias


def _add_bias(x, rdiag3):
    grid = (N_HEADS, S // 8)
    return pl.pallas_call(
        _add_bias_kernel,
        grid=grid,
        in_specs=[
            pl.BlockSpec((1, 1, M_PAD), lambda h, bi: (h, 0, 0)),
            pl.BlockSpec((1, 1, 8, S), lambda h, bi: (0, h, bi, 0)),
        ],
        out_specs=pl.BlockSpec((1, 1, 8, S), lambda h, bi: (0, h, bi, 0)),
        out_shape=jax.ShapeDtypeStruct((1, N_HEADS, S, S), jnp.float32),
        scratch_shapes=[pltpu.VMEM((8, EG_W), jnp.float32)],
    )(rdiag3, x)


@jax.jit
def kernel(x, biases):
    rdiag_t = _make_diag_table(biases)
    rdiag3 = rdiag_t.T.reshape(N_HEADS, 1, M_PAD)
    return _add_bias(x, rdiag3)


# speedup vs baseline: 55.9921x; 55.9921x over previous
import math

import jax
import jax.numpy as jnp
from jax import lax
from jax.experimental import pallas as pl
from jax.experimental.pallas import tpu as pltpu

N_BUCKETS = 32
MAX_DISTANCE = 128
N_HEADS = 16
S = 2048
M_PAD = 4352
EG_W = 4096


def _diag_table_kernel(biases_ref, out_ref):
    m = lax.broadcasted_iota(jnp.int32, (M_PAD, 1), 0)
    d = 2047 - m
    max_exact = N_BUCKETS // 2
    rp = jnp.maximum(d, 0)
    is_smol = rp < max_exact
    rp_f = jnp.maximum(rp, 1).astype(jnp.float32)
    val_if_large = max_exact + (
        jnp.log(rp_f / max_exact) / math.log(MAX_DISTANCE / max_exact)
        * (N_BUCKETS - max_exact)
    ).astype(jnp.int32)
    val_if_large = jnp.minimum(val_if_large, N_BUCKETS - 1)
    bucket = jnp.where(is_smol, rp, val_if_large)
    acc = jnp.zeros((M_PAD, N_HEADS), jnp.float32)
    for k in range(N_BUCKETS):
        acc = jnp.where(bucket == k, biases_ref[k : k + 1, :], acc)
    out_ref[:, :] = acc


def _make_diag_table(biases):
    return pl.pallas_call(
        _diag_table_kernel,
        out_shape=jax.ShapeDtypeStruct((M_PAD, N_HEADS), jnp.float32),
    )(biases)


TI = 128


def _add_bias_kernel(rdiag_ref, x_ref, out_ref, eg_ref):
    bi = pl.program_id(1)

    @pl.when(bi == 0)
    def _build_eg():
        row = rdiag_ref[0, :, :]
        for s in range(8):
            eg_ref[pl.ds(s, 1), :] = row[:, 7 - s : 7 - s + EG_W]

    base = pl.multiple_of(1920 - TI * bi, 128)
    w = eg_ref[:, pl.ds(base, S + 128)]
    for g in range(TI // 8):
        r = 8 * (15 - g)
        out_ref[0, 0, pl.ds(8 * g, 8), :] = (
            x_ref[0, 0, pl.ds(8 * g, 8), :] + w[:, r : r + S]
        )


def _add_bias(x, rdiag3):
    grid = (N_HEADS, S // TI)
    return pl.pallas_call(
        _add_bias_kernel,
        grid=grid,
        in_specs=[
            pl.BlockSpec((1, 1, M_PAD), lambda h, bi: (h, 0, 0)),
            pl.BlockSpec((1, 1, TI, S), lambda h, bi: (0, h, bi, 0)),
        ],
        out_specs=pl.BlockSpec((1, 1, TI, S), lambda h, bi: (0, h, bi, 0)),
        out_shape=jax.ShapeDtypeStruct((1, N_HEADS, S, S), jnp.float32),
        scratch_shapes=[pltpu.VMEM((8, EG_W), jnp.float32)],
    )(rdiag3, x)


@jax.jit
def kernel(x, biases):
    rdiag_t = _make_diag_table(biases)
    rdiag3 = rdiag_t.T.reshape(N_HEADS, 1, M_PAD)
    return _add_bias(x, rdiag3)
